# baseline (speedup 1.0000x reference)
import functools

import jax
import jax.numpy as jnp
from jax import lax
from jax.experimental import pallas as pl
from jax.experimental.pallas import tpu as pltpu
from jax.experimental.pallas import tpu_sc as plsc

DIM = 1024
NUM_EXPERTS = 8
HIDDEN_DIM = 4096
BATCH, SEQ = 2, 2048
TOKENS = BATCH * SEQ
CAPACITY = 320
ROWS_PER_EXPERT = BATCH * CAPACITY
ROWS = NUM_EXPERTS * ROWS_PER_EXPERT
TRASH_ROWS = 64
BUF_ROWS = ROWS + TRASH_ROWS
EPS = 1e-9
LOSS_COEF = 0.01

NUM_SC_CORES = 2
NUM_SC_SUBCORES = 16
NUM_WORKERS = NUM_SC_CORES * NUM_SC_SUBCORES
TOK_PER_WORKER = TOKENS // NUM_WORKERS
DISPATCH_CHUNK = 64
COMBINE_CHUNK = 32
LANES = 16


def _excl_cumsum(m):
    k = 16
    cn = SEQ // k
    io_i = lax.broadcasted_iota(jnp.int32, (cn, cn), 0)
    io_j = lax.broadcasted_iota(jnp.int32, (cn, cn), 1)
    ltri = (io_j < io_i).astype(jnp.float32)
    ko_i = lax.broadcasted_iota(jnp.int32, (k, k), 0)
    ko_j = lax.broadcasted_iota(jnp.int32, (k, k), 1)
    ktri = (ko_j < ko_i).astype(jnp.float32)
    chunks = [m[i * cn:(i + 1) * cn] for i in range(k)]
    sums = jnp.concatenate(
        [jnp.sum(c, axis=0, keepdims=True) for c in chunks], axis=0)
    offs = jnp.dot(ktri, sums, preferred_element_type=jnp.float32)
    parts = [
        jnp.dot(ltri, chunks[i], preferred_element_type=jnp.float32)
        + offs[i][None, :]
        for i in range(k)
    ]
    return jnp.concatenate(parts, axis=0)


def _gating_body(x_ref, wg_ref, s1_ref, s2_ref, g1_ref, g2_ref, loss_ref):
    wg = wg_ref[...]
    e = NUM_EXPERTS
    eio = lax.broadcasted_iota(jnp.int32, (SEQ, e), 1)
    nio = lax.broadcasted_iota(jnp.int32, (SEQ, 1), 0)
    loss_acc = jnp.float32(0.0)
    for b in range(BATCH):
        x = x_ref[b]
        logits = jnp.dot(x, wg, preferred_element_type=jnp.float32)
        mx = jnp.max(logits, axis=-1, keepdims=True)
        ex = jnp.exp(logits - mx)
        raw = ex / jnp.sum(ex, axis=-1, keepdims=True)

        g1 = jnp.max(raw, axis=-1, keepdims=True)
        idx1 = jnp.min(jnp.where(raw == g1, eio, e), axis=-1, keepdims=True)
        mask1 = (eio == idx1).astype(jnp.float32)
        raw2 = raw * (1.0 - mask1)
        g2 = jnp.max(raw2, axis=-1, keepdims=True)
        idx2 = jnp.min(jnp.where(raw2 == g2, eio, e), axis=-1, keepdims=True)
        mask2 = (eio == idx2).astype(jnp.float32)

        denom = g1 + g2 + EPS
        g1n = g1 / denom
        g2n = g2 / denom

        density1 = jnp.sum(mask1, axis=0, keepdims=True) / SEQ
        proxy = jnp.sum(raw, axis=0, keepdims=True) / SEQ
        loss_acc = loss_acc + jnp.sum(density1 * proxy)

        cap = jnp.float32(CAPACITY)
        pie1 = _excl_cumsum(mask1) * mask1
        keep1 = mask1 * (pie1 < cap).astype(jnp.float32)
        count1 = jnp.sum(keep1, axis=0, keepdims=True)
        flat1 = jnp.sum(keep1, axis=-1, keepdims=True)
        pos1 = jnp.sum(pie1, axis=-1, keepdims=True)

        pie2 = (_excl_cumsum(mask2) + count1) * mask2
        keep2 = mask2 * (pie2 < cap).astype(jnp.float32)
        flat2 = jnp.sum(keep2, axis=-1, keepdims=True)
        pos2 = jnp.sum(pie2, axis=-1, keepdims=True)

        g1f = g1n * flat1
        g2f = g2n * flat2

        trash = ROWS + (nio & (TRASH_ROWS - 1))
        slot1 = idx1 * ROWS_PER_EXPERT + b * CAPACITY + pos1.astype(jnp.int32)
        slot1 = jnp.where(flat1 > 0.5, slot1, trash)
        slot2 = idx2 * ROWS_PER_EXPERT + b * CAPACITY + pos2.astype(jnp.int32)
        slot2 = jnp.where(flat2 > 0.5, slot2, trash)

        sl = pl.ds(b * SEQ, SEQ)
        s1_ref[sl, :] = slot1
        s2_ref[sl, :] = slot2
        g1_ref[sl, :] = jnp.broadcast_to(g1f, (SEQ, LANES))
        g2_ref[sl, :] = jnp.broadcast_to(g2f, (SEQ, LANES))

    loss = loss_acc / (BATCH * e) * float(e * e) * LOSS_COEF
    loss_ref[...] = jnp.broadcast_to(loss, (1, 1))


_gating = pl.pallas_call(
    _gating_body,
    out_shape=[
        jax.ShapeDtypeStruct((TOKENS, 1), jnp.int32),
        jax.ShapeDtypeStruct((TOKENS, 1), jnp.int32),
        jax.ShapeDtypeStruct((TOKENS, LANES), jnp.float32),
        jax.ShapeDtypeStruct((TOKENS, LANES), jnp.float32),
        jax.ShapeDtypeStruct((1, 1), jnp.float32),
    ],
)


def _ffn_body(xe_ref, w1_ref, w2_ref, y_ref):
    h = pl.program_id(1)
    hid = jnp.dot(xe_ref[...], w1_ref[0], preferred_element_type=jnp.float32)
    hid = jnp.maximum(hid, 0.0)
    contrib = jnp.dot(hid, w2_ref[0], preferred_element_type=jnp.float32)

    @pl.when(h == 0)
    def _():
        y_ref[...] = contrib

    @pl.when(h != 0)
    def _():
        y_ref[...] += contrib


_N_HBLK = 4
_HBLK = HIDDEN_DIM // _N_HBLK

_ffn = pl.pallas_call(
    _ffn_body,
    grid=(NUM_EXPERTS, _N_HBLK),
    in_specs=[
        pl.BlockSpec((ROWS_PER_EXPERT, DIM), lambda e, h: (e, 0)),
        pl.BlockSpec((1, DIM, _HBLK), lambda e, h: (e, 0, h)),
        pl.BlockSpec((1, _HBLK, DIM), lambda e, h: (e, h, 0)),
    ],
    out_specs=pl.BlockSpec((ROWS_PER_EXPERT, DIM), lambda e, h: (e, 0)),
    out_shape=jax.ShapeDtypeStruct((ROWS, DIM), jnp.float32),
)

def _dispatch_body(x_hbm, s1_hbm, s2_hbm, buf_hbm, s1_v, s2_v, rows_v, sem):
    wid = lax.axis_index("s") * NUM_SC_CORES + lax.axis_index("c")
    base = wid * TOK_PER_WORKER
    for c in range(TOK_PER_WORKER // DISPATCH_CHUNK):
        off = base + c * DISPATCH_CHUNK
        pltpu.sync_copy(x_hbm.at[pl.ds(off, DISPATCH_CHUNK)], rows_v)
        pltpu.sync_copy(s1_hbm.at[pl.ds(off, DISPATCH_CHUNK)], s1_v)
        pltpu.sync_copy(s2_hbm.at[pl.ds(off, DISPATCH_CHUNK)], s2_v)
        pltpu.async_copy(rows_v, buf_hbm.at[s1_v], sem).wait()
        pltpu.async_copy(rows_v, buf_hbm.at[s2_v], sem).wait()


def _combine_body(y_hbm, s1_hbm, s2_hbm, g1_hbm, g2_hbm, out_hbm,
                  i1_v, i2_v, g1_v, g2_v, r1_v, r2_v, o_v, sem):
    wid = lax.axis_index("s") * NUM_SC_CORES + lax.axis_index("c")
    base = wid * TOK_PER_WORKER
    nslice = DIM // LANES
    for c in range(TOK_PER_WORKER // COMBINE_CHUNK):
        off = base + c * COMBINE_CHUNK
        pltpu.sync_copy(s1_hbm.at[pl.ds(off, COMBINE_CHUNK)], i1_v)
        pltpu.sync_copy(s2_hbm.at[pl.ds(off, COMBINE_CHUNK)], i2_v)
        pltpu.sync_copy(g1_hbm.at[pl.ds(off, COMBINE_CHUNK)], g1_v)
        pltpu.sync_copy(g2_hbm.at[pl.ds(off, COMBINE_CHUNK)], g2_v)
        for j in range(COMBINE_CHUNK // LANES):
            js = pl.ds(j * LANES, LANES)
            v1 = i1_v[js]
            i1_v[js] = jnp.where(v1 < ROWS, v1, 0)
            v2 = i2_v[js]
            i2_v[js] = jnp.where(v2 < ROWS, v2, 0)
        pltpu.async_copy(y_hbm.at[i1_v], r1_v, sem).wait()
        pltpu.async_copy(y_hbm.at[i2_v], r2_v, sem).wait()

        def token_body(t, carry):
            ga = g1_v[t]
            gb = g2_v[t]

            def slice_body(j, carry2):
                js2 = pl.ds(j * LANES, LANES)
                ca = ga * r1_v[t, js2]
                ca = jnp.where(ca == ca, ca, 0.0)
                cb = gb * r2_v[t, js2]
                cb = jnp.where(cb == cb, cb, 0.0)
                o_v[t, js2] = ca + cb
                return carry2

            return lax.fori_loop(0, nslice, slice_body, carry)

        lax.fori_loop(0, COMBINE_CHUNK, token_body, 0)
        pltpu.sync_copy(o_v, out_hbm.at[pl.ds(off, COMBINE_CHUNK)])


@functools.cache
def _sc_kernels():
    mesh = plsc.VectorSubcoreMesh(core_axis_name="c", subcore_axis_name="s")
    dispatch = pl.kernel(
        _dispatch_body,
        mesh=mesh,
        out_type=jax.ShapeDtypeStruct((BUF_ROWS, DIM), jnp.float32),
        scratch_types=[
            pltpu.VMEM((DISPATCH_CHUNK,), jnp.int32),
            pltpu.VMEM((DISPATCH_CHUNK,), jnp.int32),
            pltpu.VMEM((DISPATCH_CHUNK, DIM), jnp.float32),
            pltpu.SemaphoreType.DMA,
        ],
    )
    combine = pl.kernel(
        _combine_body,
        mesh=mesh,
        out_type=jax.ShapeDtypeStruct((TOKENS, DIM), jnp.float32),
        scratch_types=[
            pltpu.VMEM((COMBINE_CHUNK,), jnp.int32),
            pltpu.VMEM((COMBINE_CHUNK,), jnp.int32),
            pltpu.VMEM((COMBINE_CHUNK, LANES), jnp.float32),
            pltpu.VMEM((COMBINE_CHUNK, LANES), jnp.float32),
            pltpu.VMEM((COMBINE_CHUNK, DIM), jnp.float32),
            pltpu.VMEM((COMBINE_CHUNK, DIM), jnp.float32),
            pltpu.VMEM((COMBINE_CHUNK, DIM), jnp.float32),
            pltpu.SemaphoreType.DMA,
        ],
    )
    return dispatch, combine


def kernel(inputs, w_gating, w1, w2):
    dispatch, combine = _sc_kernels()
    s1, s2, g1b, g2b, loss = _gating(inputs, w_gating)
    xf = inputs.reshape(TOKENS, DIM)
    buf = dispatch(xf, s1.reshape(TOKENS), s2.reshape(TOKENS))
    y = _ffn(buf, w1, w2)
    out = combine(y, s1.reshape(TOKENS), s2.reshape(TOKENS), g1b, g2b)
    return out.reshape(BATCH, SEQ, DIM), loss[0, 0]


# speedup vs baseline: 1.0050x; 1.0050x over previous
import functools

import jax
import jax.numpy as jnp
from jax import lax
from jax.experimental import pallas as pl
from jax.experimental.pallas import tpu as pltpu
from jax.experimental.pallas import tpu_sc as plsc

DIM = 1024
NUM_EXPERTS = 8
HIDDEN_DIM = 4096
BATCH, SEQ = 2, 2048
TOKENS = BATCH * SEQ
CAPACITY = 320
ROWS_PER_EXPERT = BATCH * CAPACITY
ROWS = NUM_EXPERTS * ROWS_PER_EXPERT
TRASH_ROWS = 64
BUF_ROWS = ROWS + TRASH_ROWS
EPS = 1e-9
LOSS_COEF = 0.01

NUM_SC_CORES = 2
NUM_SC_SUBCORES = 16
NUM_WORKERS = NUM_SC_CORES * NUM_SC_SUBCORES
TOK_PER_WORKER = TOKENS // NUM_WORKERS
DISPATCH_CHUNK = 64
COMBINE_CHUNK = 32
LANES = 16


def _excl_cumsum(m):
    k = 16
    cn = SEQ // k
    io_i = lax.broadcasted_iota(jnp.int32, (cn, cn), 0)
    io_j = lax.broadcasted_iota(jnp.int32, (cn, cn), 1)
    ltri = (io_j < io_i).astype(jnp.float32)
    ko_i = lax.broadcasted_iota(jnp.int32, (k, k), 0)
    ko_j = lax.broadcasted_iota(jnp.int32, (k, k), 1)
    ktri = (ko_j < ko_i).astype(jnp.float32)
    chunks = [m[i * cn:(i + 1) * cn] for i in range(k)]
    sums = jnp.concatenate(
        [jnp.sum(c, axis=0, keepdims=True) for c in chunks], axis=0)
    offs = jnp.dot(ktri, sums, preferred_element_type=jnp.float32)
    parts = [
        jnp.dot(ltri, chunks[i], preferred_element_type=jnp.float32)
        + offs[i][None, :]
        for i in range(k)
    ]
    return jnp.concatenate(parts, axis=0)


def _gating_body(x_ref, wg_ref, s1_ref, s2_ref, g1_ref, g2_ref, loss_ref,
                 fill_ref):
    wg = wg_ref[...]
    e = NUM_EXPERTS
    eio = lax.broadcasted_iota(jnp.int32, (SEQ, e), 1)
    nio = lax.broadcasted_iota(jnp.int32, (SEQ, 1), 0)
    loss_acc = jnp.float32(0.0)
    fills = []
    for b in range(BATCH):
        x = x_ref[b]
        logits = jnp.dot(x, wg, preferred_element_type=jnp.float32)
        mx = jnp.max(logits, axis=-1, keepdims=True)
        ex = jnp.exp(logits - mx)
        raw = ex / jnp.sum(ex, axis=-1, keepdims=True)

        g1 = jnp.max(raw, axis=-1, keepdims=True)
        idx1 = jnp.min(jnp.where(raw == g1, eio, e), axis=-1, keepdims=True)
        mask1 = (eio == idx1).astype(jnp.float32)
        raw2 = raw * (1.0 - mask1)
        g2 = jnp.max(raw2, axis=-1, keepdims=True)
        idx2 = jnp.min(jnp.where(raw2 == g2, eio, e), axis=-1, keepdims=True)
        mask2 = (eio == idx2).astype(jnp.float32)

        denom = g1 + g2 + EPS
        g1n = g1 / denom
        g2n = g2 / denom

        density1 = jnp.sum(mask1, axis=0, keepdims=True) / SEQ
        proxy = jnp.sum(raw, axis=0, keepdims=True) / SEQ
        loss_acc = loss_acc + jnp.sum(density1 * proxy)

        cap = jnp.float32(CAPACITY)
        pie1 = _excl_cumsum(mask1) * mask1
        keep1 = mask1 * (pie1 < cap).astype(jnp.float32)
        count1 = jnp.sum(keep1, axis=0, keepdims=True)
        flat1 = jnp.sum(keep1, axis=-1, keepdims=True)
        pos1 = jnp.sum(pie1, axis=-1, keepdims=True)

        pie2 = (_excl_cumsum(mask2) + count1) * mask2
        keep2 = mask2 * (pie2 < cap).astype(jnp.float32)
        flat2 = jnp.sum(keep2, axis=-1, keepdims=True)
        pos2 = jnp.sum(pie2, axis=-1, keepdims=True)

        fills.append(count1 + jnp.sum(keep2, axis=0, keepdims=True))

        g1f = g1n * flat1
        g2f = g2n * flat2

        trash = ROWS + (nio & (TRASH_ROWS - 1))
        slot1 = idx1 * ROWS_PER_EXPERT + b * CAPACITY + pos1.astype(jnp.int32)
        slot1 = jnp.where(flat1 > 0.5, slot1, trash)
        slot2 = idx2 * ROWS_PER_EXPERT + b * CAPACITY + pos2.astype(jnp.int32)
        slot2 = jnp.where(flat2 > 0.5, slot2, trash)

        sl = pl.ds(b * SEQ, SEQ)
        s1_ref[sl, :] = slot1
        s2_ref[sl, :] = slot2
        g1_ref[sl, :] = jnp.broadcast_to(g1f, (SEQ, LANES))
        g2_ref[sl, :] = jnp.broadcast_to(g2f, (SEQ, LANES))

    loss = loss_acc / (BATCH * e) * float(e * e) * LOSS_COEF
    loss_ref[...] = jnp.broadcast_to(loss, (1, 1))

    rio = lax.broadcasted_iota(
        jnp.int32, (ROWS_PER_EXPERT, 1), 0).astype(jnp.float32)
    in_b0 = rio < CAPACITY
    for ei in range(e):
        f0 = fills[0][:, ei:ei + 1]
        f1 = fills[1][:, ei:ei + 1]
        col = jnp.where(in_b0, (rio < f0).astype(jnp.float32),
                        ((rio - CAPACITY) < f1).astype(jnp.float32))
        fill_ref[pl.ds(ei * ROWS_PER_EXPERT, ROWS_PER_EXPERT), :] = col


_gating = pl.pallas_call(
    _gating_body,
    out_shape=[
        jax.ShapeDtypeStruct((TOKENS, 1), jnp.int32),
        jax.ShapeDtypeStruct((TOKENS, 1), jnp.int32),
        jax.ShapeDtypeStruct((TOKENS, LANES), jnp.float32),
        jax.ShapeDtypeStruct((TOKENS, LANES), jnp.float32),
        jax.ShapeDtypeStruct((1, 1), jnp.float32),
        jax.ShapeDtypeStruct((ROWS, 1), jnp.float32),
    ],
)


def _ffn_body(xe_ref, w1_ref, w2_ref, fill_ref, y_ref):
    h = pl.program_id(1)
    hid = jnp.dot(xe_ref[...], w1_ref[0], preferred_element_type=jnp.float32)
    hid = jnp.maximum(hid, 0.0)
    contrib = jnp.dot(hid, w2_ref[0], preferred_element_type=jnp.float32)

    @pl.when(h == 0)
    def _():
        y_ref[...] = contrib

    @pl.when((h != 0) & (h != _N_HBLK - 1))
    def _():
        y_ref[...] += contrib

    @pl.when(h == _N_HBLK - 1)
    def _():
        total = y_ref[...] + contrib
        y_ref[...] = jnp.where(fill_ref[...] > 0.5, total, 0.0)


_N_HBLK = 4
_HBLK = HIDDEN_DIM // _N_HBLK

_ffn = pl.pallas_call(
    _ffn_body,
    grid=(NUM_EXPERTS, _N_HBLK),
    in_specs=[
        pl.BlockSpec((ROWS_PER_EXPERT, DIM), lambda e, h: (e, 0)),
        pl.BlockSpec((1, DIM, _HBLK), lambda e, h: (e, 0, h)),
        pl.BlockSpec((1, _HBLK, DIM), lambda e, h: (e, h, 0)),
        pl.BlockSpec((ROWS_PER_EXPERT, 1), lambda e, h: (e, 0)),
    ],
    out_specs=pl.BlockSpec((ROWS_PER_EXPERT, DIM), lambda e, h: (e, 0)),
    out_shape=jax.ShapeDtypeStruct((ROWS, DIM), jnp.float32),
)

def _dispatch_body(x_hbm, s1_hbm, s2_hbm, buf_hbm, s1_v, s2_v, rows_v, sem):
    wid = lax.axis_index("s") * NUM_SC_CORES + lax.axis_index("c")
    base = wid * TOK_PER_WORKER
    for c in range(TOK_PER_WORKER // DISPATCH_CHUNK):
        off = base + c * DISPATCH_CHUNK
        pltpu.sync_copy(x_hbm.at[pl.ds(off, DISPATCH_CHUNK)], rows_v)
        pltpu.sync_copy(s1_hbm.at[pl.ds(off, DISPATCH_CHUNK)], s1_v)
        pltpu.sync_copy(s2_hbm.at[pl.ds(off, DISPATCH_CHUNK)], s2_v)
        pltpu.async_copy(rows_v, buf_hbm.at[s1_v], sem).wait()
        pltpu.async_copy(rows_v, buf_hbm.at[s2_v], sem).wait()


def _combine_body(y_hbm, s1_hbm, s2_hbm, g1_hbm, g2_hbm, out_hbm,
                  i1_v, i2_v, g1_v, g2_v, r1_v, r2_v, o_v, sem):
    wid = lax.axis_index("s") * NUM_SC_CORES + lax.axis_index("c")
    base = wid * TOK_PER_WORKER
    nslice = DIM // LANES
    for c in range(TOK_PER_WORKER // COMBINE_CHUNK):
        off = base + c * COMBINE_CHUNK
        pltpu.sync_copy(s1_hbm.at[pl.ds(off, COMBINE_CHUNK)], i1_v)
        pltpu.sync_copy(s2_hbm.at[pl.ds(off, COMBINE_CHUNK)], i2_v)
        pltpu.sync_copy(g1_hbm.at[pl.ds(off, COMBINE_CHUNK)], g1_v)
        pltpu.sync_copy(g2_hbm.at[pl.ds(off, COMBINE_CHUNK)], g2_v)
        for j in range(COMBINE_CHUNK // LANES):
            js = pl.ds(j * LANES, LANES)
            v1 = i1_v[js]
            i1_v[js] = jnp.where(v1 < ROWS, v1, 0)
            v2 = i2_v[js]
            i2_v[js] = jnp.where(v2 < ROWS, v2, 0)
        pltpu.async_copy(y_hbm.at[i1_v], r1_v, sem).wait()
        pltpu.async_copy(y_hbm.at[i2_v], r2_v, sem).wait()

        def token_body(t, carry):
            ga = g1_v[t]
            gb = g2_v[t]
            for j in range(nslice):
                js2 = pl.ds(j * LANES, LANES)
                o_v[t, js2] = ga * r1_v[t, js2] + gb * r2_v[t, js2]
            return carry

        lax.fori_loop(0, COMBINE_CHUNK, token_body, 0)
        pltpu.sync_copy(o_v, out_hbm.at[pl.ds(off, COMBINE_CHUNK)])


@functools.cache
def _sc_kernels():
    mesh = plsc.VectorSubcoreMesh(core_axis_name="c", subcore_axis_name="s")
    dispatch = pl.kernel(
        _dispatch_body,
        mesh=mesh,
        out_type=jax.ShapeDtypeStruct((BUF_ROWS, DIM), jnp.float32),
        scratch_types=[
            pltpu.VMEM((DISPATCH_CHUNK,), jnp.int32),
            pltpu.VMEM((DISPATCH_CHUNK,), jnp.int32),
            pltpu.VMEM((DISPATCH_CHUNK, DIM), jnp.float32),
            pltpu.SemaphoreType.DMA,
        ],
    )
    combine = pl.kernel(
        _combine_body,
        mesh=mesh,
        out_type=jax.ShapeDtypeStruct((TOKENS, DIM), jnp.float32),
        scratch_types=[
            pltpu.VMEM((COMBINE_CHUNK,), jnp.int32),
            pltpu.VMEM((COMBINE_CHUNK,), jnp.int32),
            pltpu.VMEM((COMBINE_CHUNK, LANES), jnp.float32),
            pltpu.VMEM((COMBINE_CHUNK, LANES), jnp.float32),
            pltpu.VMEM((COMBINE_CHUNK, DIM), jnp.float32),
            pltpu.VMEM((COMBINE_CHUNK, DIM), jnp.float32),
            pltpu.VMEM((COMBINE_CHUNK, DIM), jnp.float32),
            pltpu.SemaphoreType.DMA,
        ],
    )
    return dispatch, combine


def kernel(inputs, w_gating, w1, w2):
    dispatch, combine = _sc_kernels()
    s1, s2, g1b, g2b, loss, fill = _gating(inputs, w_gating)
    xf = inputs.reshape(TOKENS, DIM)
    buf = dispatch(xf, s1.reshape(TOKENS), s2.reshape(TOKENS))
    y = _ffn(buf, w1, w2, fill)
    out = combine(y, s1.reshape(TOKENS), s2.reshape(TOKENS), g1b, g2b)
    return out.reshape(BATCH, SEQ, DIM), loss[0, 0]
